# Initial kernel scaffold; baseline (speedup 1.0000x reference)
import jax
import jax.numpy as jnp
from jax.experimental import pallas as pl


def kernel(x, edge_index, edge_weight, W1, b1, W2, b2, W3, b3, P1, bp1, P2, bp2):
    raise NotImplementedError("write your pallas kernel here")


# speedup vs baseline: 2.7055x; 2.7055x over previous
import functools

import jax
import jax.numpy as jnp
from jax import lax
from jax.experimental import pallas as pl
from jax.experimental.pallas import tpu as pltpu
from jax.experimental.pallas import tpu_sc as plsc

N = 10000
E = 160000
D = 256
H = 128
NT = 16
EPT = E // NT
CH = 80
NCH = EPT // CH
RPT = 624
REM = N - NT * RPT
ZR = 208

_mesh = plsc.VectorSubcoreMesh(core_axis_name="c", subcore_axis_name="s")


@functools.partial(
    pl.kernel,
    mesh=_mesh,
    out_type=jax.ShapeDtypeStruct((2 * N, H), jnp.float32),
    scratch_types=[
        pltpu.VMEM_SHARED((N, H), jnp.float32),
        pltpu.VMEM((ZR, H), jnp.float32),
        pltpu.VMEM((CH,), jnp.int32),
        pltpu.VMEM((CH,), jnp.int32),
        pltpu.VMEM((CH,), jnp.float32),
        pltpu.VMEM((CH, H), jnp.float32),
        pltpu.SemaphoreType.DMA,
    ],
)
def _sc_propagate(h_stack, src_hbm, dst_hbm, w_hbm,
                  o_stack, acc, zbuf, src_v, dst_v, w_v, rows, sem):
    c = lax.axis_index("c")
    s = lax.axis_index("s")
    half = c * N

    zv = jnp.zeros((16,), jnp.float32)

    def _zrow(j, carry):
        for k in range(H // 16):
            zbuf[j, pl.ds(k * 16, 16)] = zv
        return carry

    lax.fori_loop(0, ZR, _zrow, 0)
    for q in range(RPT // ZR):
        pltpu.sync_copy(zbuf, acc.at[pl.ds(s * RPT + q * ZR, ZR)])

    @pl.when(s == 0)
    def _():
        pltpu.sync_copy(zbuf.at[pl.ds(0, REM)], acc.at[pl.ds(NT * RPT, REM)])

    plsc.subcore_barrier()

    base = s * EPT

    def _chunk(i, carry):
        off = base + i * CH
        pltpu.sync_copy(src_hbm.at[pl.ds(off, CH)], src_v)
        pltpu.sync_copy(dst_hbm.at[pl.ds(off, CH)], dst_v)
        pltpu.sync_copy(w_hbm.at[pl.ds(off, CH)], w_v)

        hv = jnp.full((16,), half, jnp.int32)
        for q in range(CH // 16):
            sl = pl.ds(q * 16, 16)
            src_v[sl] = src_v[sl] + hv

        pltpu.async_copy(h_stack.at[src_v], rows, sem).wait()

        def _scale(q, inner):
            wvec = w_v[pl.ds(q * 16, 16)]
            for j in range(16):
                wspl = wvec.at[jnp.full((16,), j, jnp.int32)].get(
                    mode="promise_in_bounds")
                r = q * 16 + j
                for k in range(H // 16):
                    sl = pl.ds(k * 16, 16)
                    rows[r, sl] = rows[r, sl] * wspl
            return inner

        lax.fori_loop(0, CH // 16, _scale, 0)
        pltpu.sync_copy(rows, acc.at[dst_v], add=True)
        return carry

    lax.fori_loop(0, NCH, _chunk, 0)
    plsc.subcore_barrier()

    r0 = s * RPT
    pltpu.sync_copy(acc.at[pl.ds(r0, RPT)], o_stack.at[pl.ds(half + r0, RPT)])

    @pl.when(s == 0)
    def _():
        pltpu.sync_copy(acc.at[pl.ds(NT * RPT, REM)],
                        o_stack.at[pl.ds(half + NT * RPT, REM)])


BR = 2000
G = N // BR

_f32 = jnp.float32
_sds = jax.ShapeDtypeStruct


def _mm_first_body(x_ref, w_ref, b_ref, o_ref):
    o_ref[...] = (jnp.dot(x_ref[...], w_ref[...], preferred_element_type=_f32)
                  + b_ref[...])


def _mm_mid_body(lo_ref, hi_ref, w_ref, b_ref, o_ref):
    x = jnp.maximum(jnp.concatenate([lo_ref[...], hi_ref[...]], axis=1), 0.0)
    o_ref[...] = jnp.dot(x, w_ref[...], preferred_element_type=_f32) + b_ref[...]


def _proj_body(lo_ref, hi_ref, p1_ref, bp1_ref, p2_ref, bp2_ref, z_ref, h_ref):
    hcat = jnp.concatenate([lo_ref[...], hi_ref[...]], axis=1)
    h_ref[...] = hcat
    t = jnp.maximum(
        jnp.dot(hcat, p1_ref[...], preferred_element_type=_f32) + bp1_ref[...],
        0.0)
    z_ref[...] = jnp.dot(t, p2_ref[...], preferred_element_type=_f32) + bp2_ref[...]


_x_spec = pl.BlockSpec((BR, D), lambda i, j: (i, 0))
_whalf_spec = pl.BlockSpec((D, H), lambda i, j: (0, j))
_bhalf_spec = pl.BlockSpec((1, H), lambda i, j: (0, j))
_stack_out_spec = pl.BlockSpec((BR, H), lambda i, j: (j * G + i, 0))
_lo_spec = pl.BlockSpec((BR, H), lambda i, j: (i, 0))
_hi_spec = pl.BlockSpec((BR, H), lambda i, j: (G + i, 0))
_stack_sds = _sds((2 * N, H), _f32)


def _mm_first(x, W, b):
    return pl.pallas_call(
        _mm_first_body, grid=(G, 2),
        in_specs=[_x_spec, _whalf_spec, _bhalf_spec],
        out_specs=_stack_out_spec,
        out_shape=_stack_sds,
    )(x, W, b.reshape(1, D))


def _mm_mid(g, W, b):
    return pl.pallas_call(
        _mm_mid_body, grid=(G, 2),
        in_specs=[_lo_spec, _hi_spec, _whalf_spec, _bhalf_spec],
        out_specs=_stack_out_spec,
        out_shape=_stack_sds,
    )(g, g, W, b.reshape(1, D))


def _proj(g, P1, bp1, P2, bp2):
    row_spec = pl.BlockSpec((BR, D), lambda i: (i, 0))
    lo = pl.BlockSpec((BR, H), lambda i: (i, 0))
    hi = pl.BlockSpec((BR, H), lambda i: (G + i, 0))
    w_spec = pl.BlockSpec((D, D), lambda i: (0, 0))
    b_spec = pl.BlockSpec((1, D), lambda i: (0, 0))
    return pl.pallas_call(
        _proj_body, grid=(G,),
        in_specs=[lo, hi, w_spec, b_spec, w_spec, b_spec],
        out_specs=[row_spec, row_spec],
        out_shape=[_sds((N, D), _f32), _sds((N, D), _f32)],
    )(g, g, P1, bp1.reshape(1, D), P2, bp2.reshape(1, D))


def kernel(x, edge_index, edge_weight, W1, b1, W2, b2, W3, b3, P1, bp1, P2, bp2):
    src = edge_index[0].astype(jnp.int32)
    dst = edge_index[1].astype(jnp.int32)
    w = edge_weight.astype(jnp.float32)

    h = _mm_first(x, W1, b1)
    g = _sc_propagate(h, src, dst, w)
    h = _mm_mid(g, W2, b2)
    g = _sc_propagate(h, src, dst, w)
    h = _mm_mid(g, W3, b3)
    g = _sc_propagate(h, src, dst, w)
    z, hout = _proj(g, P1, bp1, P2, bp2)
    return (z, hout)


# speedup vs baseline: 2.7538x; 1.0178x over previous
import functools

import jax
import jax.numpy as jnp
from jax import lax
from jax.experimental import pallas as pl
from jax.experimental.pallas import tpu as pltpu
from jax.experimental.pallas import tpu_sc as plsc

N = 10000
E = 160000
D = 256
H = 128
NT = 16
EPT = E // NT
CH = 80
NCH = EPT // CH
NCHP = 128
PH = NCHP // 2
RPT = 624
REM = N - NT * RPT

_mesh = plsc.VectorSubcoreMesh(core_axis_name="c", subcore_axis_name="s")


@functools.partial(
    pl.kernel,
    mesh=_mesh,
    out_type=jax.ShapeDtypeStruct((2 * N, H), jnp.float32),
    scratch_types=[
        pltpu.VMEM_SHARED((N, H), jnp.float32),
        pltpu.VMEM((PH, CH), jnp.int32),
        pltpu.VMEM((PH, CH), jnp.int32),
        pltpu.VMEM((PH, CH), jnp.float32),
        pltpu.VMEM((CH, H), jnp.float32),
        pltpu.VMEM((CH, H), jnp.float32),
        pltpu.SemaphoreType.DMA,
        pltpu.SemaphoreType.DMA,
        pltpu.SemaphoreType.DMA,
        pltpu.SemaphoreType.DMA,
    ],
)
def _sc_propagate(h_stack, src_hbm, dst_hbm, w_hbm, o_stack,
                  acc, src_t, dst_t, w_t, rows_a, rows_b,
                  sem_ga, sem_gb, sem_sa, sem_sb):
    c = lax.axis_index("c")
    s = lax.axis_index("s")
    half = c * N

    zv = jnp.zeros((16,), jnp.float32)

    def _zrow(j, carry):
        for k in range(H // 16):
            rows_a[j, pl.ds(k * 16, 16)] = zv
        return carry

    lax.fori_loop(0, CH, _zrow, 0)

    for q in range(RPT // CH):
        pltpu.sync_copy(rows_a, acc.at[pl.ds(s * RPT + q * CH, CH)])
    pltpu.sync_copy(rows_a.at[pl.ds(0, RPT % CH)],
                    acc.at[pl.ds(s * RPT + (RPT // CH) * CH, RPT % CH)])

    @pl.when(s == 0)
    def _():
        pltpu.sync_copy(rows_a.at[pl.ds(0, REM)], acc.at[pl.ds(NT * RPT, REM)])

    plsc.subcore_barrier()

    def _fire_gather(i, buf, sem):
        pltpu.async_copy(h_stack.at[src_t.at[i]], buf, sem)

    def _wait(buf, sem):
        pltpu.make_async_copy(h_stack.at[pl.ds(0, CH)], buf, sem).wait()

    def _scale(i, buf):
        def _grp(q, inner):
            wvec = w_t[i, pl.ds(q * 16, 16)]
            for j in range(16):
                wspl = wvec.at[jnp.full((16,), j, jnp.int32)].get(
                    mode="promise_in_bounds")
                r = q * 16 + j
                for k in range(H // 16):
                    sl = pl.ds(k * 16, 16)
                    buf[r, sl] = buf[r, sl] * wspl
            return inner

        lax.fori_loop(0, CH // 16, _grp, 0)

    def _fire_scatter(i, buf, sem):
        pltpu.async_copy(buf, acc.at[dst_t.at[i]], sem, add=True)

    def _wait_scatter(buf, sem):
        pltpu.make_async_copy(buf, acc.at[pl.ds(0, CH)], sem).wait()

    hv = jnp.full((16,), half, jnp.int32)

    def _adj(j, carry):
        for q in range(CH // 16):
            sl = pl.ds(q * 16, 16)
            src_t[j, sl] = src_t[j, sl] + hv
        return carry

    def _pair(p, carry):
        i = 2 * p
        _wait(rows_a, sem_ga)
        _scale(i, rows_a)
        _fire_scatter(i, rows_a, sem_sa)
        _wait(rows_b, sem_gb)
        _scale(i + 1, rows_b)
        _fire_scatter(i + 1, rows_b, sem_sb)
        _wait_scatter(rows_a, sem_sa)
        _fire_gather(jnp.minimum(i + 2, PH - 1), rows_a, sem_ga)
        _wait_scatter(rows_b, sem_sb)
        _fire_gather(jnp.minimum(i + 3, PH - 1), rows_b, sem_gb)
        return carry

    for ph in range(NCHP // PH):
        pltpu.sync_copy(src_hbm.at[s, pl.ds(ph * PH, PH)], src_t)
        pltpu.sync_copy(dst_hbm.at[s, pl.ds(ph * PH, PH)], dst_t)
        pltpu.sync_copy(w_hbm.at[s, pl.ds(ph * PH, PH)], w_t)
        lax.fori_loop(0, PH, _adj, 0)
        _fire_gather(0, rows_a, sem_ga)
        _fire_gather(1, rows_b, sem_gb)
        lax.fori_loop(0, PH // 2, _pair, 0)
        _wait(rows_a, sem_ga)
        _wait(rows_b, sem_gb)

    plsc.subcore_barrier()

    r0 = s * RPT
    pltpu.sync_copy(acc.at[pl.ds(r0, RPT)], o_stack.at[pl.ds(half + r0, RPT)])

    @pl.when(s == 0)
    def _():
        pltpu.sync_copy(acc.at[pl.ds(NT * RPT, REM)],
                        o_stack.at[pl.ds(half + NT * RPT, REM)])


BR = 2000
G = N // BR

_f32 = jnp.float32
_sds = jax.ShapeDtypeStruct


def _mm_first_body(x_ref, w_ref, b_ref, o_ref):
    o_ref[...] = (jnp.dot(x_ref[...], w_ref[...], preferred_element_type=_f32)
                  + b_ref[...])


def _mm_mid_body(lo_ref, hi_ref, w_ref, b_ref, o_ref):
    x = jnp.maximum(jnp.concatenate([lo_ref[...], hi_ref[...]], axis=1), 0.0)
    o_ref[...] = jnp.dot(x, w_ref[...], preferred_element_type=_f32) + b_ref[...]


def _proj_body(lo_ref, hi_ref, p1_ref, bp1_ref, p2_ref, bp2_ref, z_ref, h_ref):
    hcat = jnp.concatenate([lo_ref[...], hi_ref[...]], axis=1)
    h_ref[...] = hcat
    t = jnp.maximum(
        jnp.dot(hcat, p1_ref[...], preferred_element_type=_f32) + bp1_ref[...],
        0.0)
    z_ref[...] = jnp.dot(t, p2_ref[...], preferred_element_type=_f32) + bp2_ref[...]


_x_spec = pl.BlockSpec((BR, D), lambda i, j: (i, 0))
_whalf_spec = pl.BlockSpec((D, H), lambda i, j: (0, j))
_bhalf_spec = pl.BlockSpec((1, H), lambda i, j: (0, j))
_stack_out_spec = pl.BlockSpec((BR, H), lambda i, j: (j * G + i, 0))
_lo_spec = pl.BlockSpec((BR, H), lambda i, j: (i, 0))
_hi_spec = pl.BlockSpec((BR, H), lambda i, j: (G + i, 0))
_stack_sds = _sds((2 * N, H), _f32)


def _mm_first(x, W, b):
    return pl.pallas_call(
        _mm_first_body, grid=(G, 2),
        in_specs=[_x_spec, _whalf_spec, _bhalf_spec],
        out_specs=_stack_out_spec,
        out_shape=_stack_sds,
    )(x, W, b.reshape(1, D))


def _mm_mid(g, W, b):
    return pl.pallas_call(
        _mm_mid_body, grid=(G, 2),
        in_specs=[_lo_spec, _hi_spec, _whalf_spec, _bhalf_spec],
        out_specs=_stack_out_spec,
        out_shape=_stack_sds,
    )(g, g, W, b.reshape(1, D))


def _proj(g, P1, bp1, P2, bp2):
    row_spec = pl.BlockSpec((BR, D), lambda i: (i, 0))
    lo = pl.BlockSpec((BR, H), lambda i: (i, 0))
    hi = pl.BlockSpec((BR, H), lambda i: (G + i, 0))
    w_spec = pl.BlockSpec((D, D), lambda i: (0, 0))
    b_spec = pl.BlockSpec((1, D), lambda i: (0, 0))
    return pl.pallas_call(
        _proj_body, grid=(G,),
        in_specs=[lo, hi, w_spec, b_spec, w_spec, b_spec],
        out_specs=[row_spec, row_spec],
        out_shape=[_sds((N, D), _f32), _sds((N, D), _f32)],
    )(g, g, P1, bp1.reshape(1, D), P2, bp2.reshape(1, D))


def kernel(x, edge_index, edge_weight, W1, b1, W2, b2, W3, b3, P1, bp1, P2, bp2):
    pad = ((0, 0), (0, NCHP - NCH), (0, 0))
    src = jnp.pad(edge_index[0].astype(jnp.int32).reshape(NT, NCH, CH), pad)
    dst = jnp.pad(edge_index[1].astype(jnp.int32).reshape(NT, NCH, CH), pad)
    w = jnp.pad(edge_weight.astype(jnp.float32).reshape(NT, NCH, CH), pad)

    h = _mm_first(x, W1, b1)
    g = _sc_propagate(h, src, dst, w)
    h = _mm_mid(g, W2, b2)
    g = _sc_propagate(h, src, dst, w)
    h = _mm_mid(g, W3, b3)
    g = _sc_propagate(h, src, dst, w)
    z, hout = _proj(g, P1, bp1, P2, bp2)
    return (z, hout)
